# baseline (speedup 1.0000x reference)
import functools

import jax
import jax.numpy as jnp
from jax import lax
from jax.experimental import pallas as pl
from jax.experimental.pallas import tpu as pltpu
from jax.experimental.pallas import tpu_sc as plsc

_NUM_TYPES = 100000
_EMBED_DIM = 64
_BATCH = 16384

_INFO = plsc.get_sparse_core_info()
_NC = _INFO.num_cores
_NS = _INFO.num_subcores
_NW = _NC * _NS
_B_PER_W = _BATCH // _NW
_RING = 16


@functools.partial(
    pl.kernel,
    mesh=plsc.VectorSubcoreMesh(core_axis_name="c", subcore_axis_name="s"),
    out_type=jax.ShapeDtypeStruct((_BATCH // 8, 8, _EMBED_DIM), jnp.float32),
    scratch_types=[
        pltpu.VMEM((_B_PER_W,), jnp.int32),
        pltpu.VMEM((_B_PER_W // 8, 8, _EMBED_DIM), jnp.float32),
        pltpu.SemaphoreType.DMA,
    ],
)
def _embed_gather(table_hbm, idx_hbm, out_hbm, idx_s, rows_v, sem):
    wid = lax.axis_index("s") * _NC + lax.axis_index("c")
    base = wid * _B_PER_W
    pltpu.sync_copy(idx_hbm.at[pl.ds(base, _B_PER_W)], idx_s)

    def wait_eight_rows():
        pltpu.make_async_copy(
            table_hbm.at[0],
            rows_v.at[0],
            sem,
        ).wait()

    def fire_group(g):
        iv = idx_s[pl.ds(g * 16, 16)]
        for k in range(16):
            r = iv[k]
            pltpu.async_copy(
                table_hbm.at[r >> 3, pl.ds(r & 7, 1)],
                rows_v.at[2 * g + k // 8, pl.ds(k % 8, 1)],
                sem,
            )

    fire_group(0)

    def body(g, _):
        fire_group(g)
        wait_eight_rows()
        wait_eight_rows()
        return 0

    lax.fori_loop(1, _B_PER_W // 16, body, 0)
    wait_eight_rows()
    wait_eight_rows()
    pltpu.sync_copy(rows_v, out_hbm.at[pl.ds(wid * (_B_PER_W // 8), _B_PER_W // 8)])


def kernel(disaster_type_idx, embedding_weight):
    idx = disaster_type_idx.astype(jnp.int32)
    table3 = embedding_weight.reshape(_NUM_TYPES // 8, 8, _EMBED_DIM)
    out3 = _embed_gather(table3, idx)
    return out3.reshape(_BATCH, _EMBED_DIM)
